# Initial kernel scaffold; baseline (speedup 1.0000x reference)
import jax
import jax.numpy as jnp
from jax.experimental import pallas as pl


def kernel(x, all_signs, wht_mat, centroids):
    raise NotImplementedError("write your pallas kernel here")


# speedup vs baseline: 1171.5491x; 1171.5491x over previous
import functools

import jax
import jax.numpy as jnp
from jax.experimental import pallas as pl
from jax.experimental.pallas import tpu as pltpu

_DIM = 64
_EPS = 1e-8
_FINAL_SCALE = _DIM ** -0.5
_NLEV = 256

_K_LUT = 512
_LUT_LO = -5.12
_LUT_STEP = 10.24 / _K_LUT
_LUT_INV_STEP = 1.0 / _LUT_STEP
_V_CLIP = 5.1
_BIG = 1e30

_ROWS_PER_BLOCK = 4096


def _lut_lookup(bnext_ref, clo_ref, chi_ref, v):
  vc = jnp.clip(v, -_V_CLIP, _V_CLIP)
  k = ((vc - _LUT_LO) * _LUT_INV_STEP).astype(jnp.int32)
  k7 = jnp.bitwise_and(k, 127)
  kh = jnp.right_shift(k, 7)
  shape = v.shape[:-1] + (128,)
  bnext = jnp.zeros(v.shape, jnp.float32)
  clo = jnp.zeros(v.shape, jnp.float32)
  chi = jnp.zeros(v.shape, jnp.float32)
  for g in range(4):
    sel = kh == g
    tb = jnp.broadcast_to(bnext_ref[g:g + 1, :], shape)
    tl = jnp.broadcast_to(clo_ref[g:g + 1, :], shape)
    th = jnp.broadcast_to(chi_ref[g:g + 1, :], shape)
    bnext = jnp.where(sel, jnp.take_along_axis(tb, k7, axis=-1), bnext)
    clo = jnp.where(sel, jnp.take_along_axis(tl, k7, axis=-1), clo)
    chi = jnp.where(sel, jnp.take_along_axis(th, k7, axis=-1), chi)
  return jnp.where(vc > bnext, chi, clo)


def _quant_kernel(x_ref, a_ref, b_ref, bnext_ref, clo_ref, chi_ref,
                  maxc_ref, out_ref):
  xf = x_ref[...]
  norms = jnp.sqrt(jnp.sum(xf * xf, axis=-1, keepdims=True)) + _EPS
  xu = xf / norms
  x_rot = jnp.dot(xu, a_ref[...], preferred_element_type=jnp.float32)

  max_c = maxc_ref[0, 0]
  x_rot_max = jnp.max(jnp.abs(x_rot), axis=-1, keepdims=True)
  rms_scales = x_rot_max / max_c
  x_normalized = x_rot / (rms_scales + _EPS)

  recon_u = _lut_lookup(bnext_ref, clo_ref, chi_ref, x_normalized)
  num = jnp.sum(x_rot * recon_u, axis=-1, keepdims=True)
  den = jnp.sum(recon_u * recon_u, axis=-1, keepdims=True) + _EPS
  gamma1 = num / den
  recon_2 = _lut_lookup(bnext_ref, clo_ref, chi_ref, x_rot / (gamma1 + _EPS))

  mean_abs = jnp.mean(jnp.abs(x_rot), axis=-1, keepdims=True) + _EPS
  is_spiky = (x_rot_max / mean_abs) > 5.0
  gamma = jnp.where(is_spiky, rms_scales, gamma1)
  recon = jnp.where(is_spiky, recon_u, recon_2) * gamma

  x_unit = jnp.dot(recon, b_ref[...], preferred_element_type=jnp.float32)
  out_ref[...] = x_unit * norms


@jax.jit
def kernel(x, all_signs, wht_mat, centroids):
  shape = x.shape
  xf = x.astype(jnp.float32).reshape(-1, _DIM)
  n_rows = xf.shape[0]

  signs = all_signs[0]
  a_mat = (signs[:, None] * wht_mat) * _FINAL_SCALE
  b_mat = (wht_mat * signs[None, :]) * _FINAL_SCALE

  boundaries = (centroids[:-1] + centroids[1:]) * 0.5
  edges = _LUT_LO + _LUT_STEP * jnp.arange(_K_LUT, dtype=jnp.float32)
  base = jnp.searchsorted(boundaries, edges, side='left').astype(jnp.int32)
  bnext = jnp.where(base < _NLEV - 1,
                    boundaries[jnp.minimum(base, _NLEV - 2)], _BIG)
  clo = centroids[base]
  chi = centroids[jnp.minimum(base + 1, _NLEV - 1)]
  bnext = bnext.reshape(4, 128)
  clo = clo.reshape(4, 128)
  chi = chi.reshape(4, 128)
  maxc = centroids[-1].reshape(1, 1)

  rows_blk = min(_ROWS_PER_BLOCK, n_rows)
  grid = (n_rows // rows_blk,)

  full = lambda s: pl.BlockSpec(s, lambda i: (0, 0))
  out = pl.pallas_call(
      _quant_kernel,
      grid=grid,
      in_specs=[
          pl.BlockSpec((rows_blk, _DIM), lambda i: (i, 0)),
          full((_DIM, _DIM)),
          full((_DIM, _DIM)),
          full((4, 128)),
          full((4, 128)),
          full((4, 128)),
          full((1, 1)),
      ],
      out_specs=pl.BlockSpec((rows_blk, _DIM), lambda i: (i, 0)),
      out_shape=jax.ShapeDtypeStruct((n_rows, _DIM), jnp.float32),
  )(xf, a_mat, b_mat, bnext, clo, chi, maxc)
  return out.reshape(shape)
